# baseline (speedup 1.0000x reference)
import functools

import jax
import jax.numpy as jnp
import numpy as np
from jax.experimental import pallas as pl
from jax.experimental.pallas import tpu as pltpu

_NG = 4
_D = 32
_E = 8
_OC = 16
_B = 64
_L = 4096
_LO = _L - 2
_GD = _NG * _OC


def _cv2(v):
    mean = jnp.sum(v, axis=1, keepdims=True) / _E
    var = jnp.sum((v - mean) ** 2, axis=1, keepdims=True) / (_E - 1)
    return var / (mean * mean + 1e-10)


def _gate_body(gx_ref, wg_ref, gates_ref, loss_ref):
    iota = jax.lax.broadcasted_iota(jnp.int32, (_B, _E), 1)
    loss = jnp.zeros((1, 1), jnp.float32)
    for g in range(_NG):
        logits = jnp.dot(gx_ref[g], wg_ref[g],
                         preferred_element_type=jnp.float32)
        p = jax.nn.softmax(logits, axis=1)
        m1 = jnp.max(p, axis=1, keepdims=True)
        i1 = jnp.min(jnp.where(p == m1, iota, _E), axis=1, keepdims=True)
        mask1 = iota == i1
        p2 = jnp.where(mask1, -jnp.inf, p)
        m2 = jnp.max(p2, axis=1, keepdims=True)
        i2 = jnp.min(jnp.where(p2 == m2, iota, _E), axis=1, keepdims=True)
        denom = m1 + m2 + 1e-6
        gates = jnp.where(mask1, m1 / denom, 0.0) + jnp.where(
            iota == i2, m2 / denom, 0.0)
        gates_ref[g] = gates
        imp = jnp.sum(gates, axis=0, keepdims=True)
        load = jnp.sum((gates > 0.0).astype(jnp.float32), axis=0,
                       keepdims=True)
        loss = loss + (_cv2(imp) + _cv2(load)) * 0.01
    loss_ref[...] = loss


_BB = 8


def _main_body(x_ref, w1_ref, b1_ref, w2blk_ref, b2f_ref, gsel_ref, g_ref,
               out_ref):
    for bb in range(_BB):
        x = x_ref[bb]
        y0 = jnp.dot(w1_ref[0], x, preferred_element_type=jnp.float32)
        y1 = jnp.dot(w1_ref[1], x, preferred_element_type=jnp.float32)
        y2 = jnp.dot(w1_ref[2], x, preferred_element_type=jnp.float32)
        acc = (y0[:, 0:_LO] + y1[:, 1:1 + _LO] + y2[:, 2:2 + _LO]
               + b1_ref[...])
        h = jnp.tanh(acc)
        gb = g_ref[bb]
        ge = jnp.dot(gsel_ref[...], gb, preferred_element_type=jnp.float32)
        w2bd = jnp.zeros((_GD, _GD), jnp.float32)
        for e in range(_E):
            w2bd = w2bd + w2blk_ref[e] * ge[:, e:e + 1]
        b2e = jnp.sum(b2f_ref[...] * ge, axis=1, keepdims=True)
        out = jnp.dot(w2bd, h, preferred_element_type=jnp.float32) + b2e
        out_ref[bb] = out


@functools.partial(jax.jit, static_argnames=("interpret",))
def _run(x, Wg, W1, b1, W2, b2, interpret=False):
    f32 = jnp.float32
    xs = jax.lax.slice_in_dim(x, _L - 6, _L - 1, axis=2)
    gx = xs.reshape(_B, _NG, _D * 5).transpose(1, 0, 2)

    gates, loss = pl.pallas_call(
        _gate_body,
        out_shape=(
            jax.ShapeDtypeStruct((_NG, _B, _E), f32),
            jax.ShapeDtypeStruct((1, 1), f32),
        ),
        interpret=interpret,
    )(gx, Wg)

    w1bd = jnp.zeros((3, _GD, _NG * _D), f32)
    for g in range(_NG):
        w1bd = jax.lax.dynamic_update_slice(
            w1bd, W1[g].transpose(2, 0, 1), (0, g * _OC, g * _D))
    b1f = b1.reshape(_GD, 1)
    w2r = W2[:, :, :, 0].reshape(_NG, _OC, _E, _OC)
    w2blk = jnp.zeros((_E, _GD, _GD), f32)
    for g in range(_NG):
        w2blk = jax.lax.dynamic_update_slice(
            w2blk, w2r[g].transpose(1, 0, 2), (0, g * _OC, g * _OC))
    b2f = b2.reshape(_NG, _OC, _E).reshape(_GD, _E)
    gsel = jnp.repeat(jnp.eye(_NG, dtype=f32), _OC, axis=0)
    gates_t = gates.transpose(1, 0, 2)

    out = pl.pallas_call(
        _main_body,
        grid=(_B // _BB,),
        in_specs=[
            pl.BlockSpec((_BB, _NG * _D, _L), lambda b: (b, 0, 0)),
            pl.BlockSpec((3, _GD, _NG * _D), lambda b: (0, 0, 0)),
            pl.BlockSpec((_GD, 1), lambda b: (0, 0)),
            pl.BlockSpec((_E, _GD, _GD), lambda b: (0, 0, 0)),
            pl.BlockSpec((_GD, _E), lambda b: (0, 0)),
            pl.BlockSpec((_GD, _NG), lambda b: (0, 0)),
            pl.BlockSpec((_BB, _NG, _E), lambda b: (b, 0, 0)),
        ],
        out_specs=pl.BlockSpec((_BB, _GD, _LO), lambda b: (b, 0, 0)),
        out_shape=jax.ShapeDtypeStruct((_B, _GD, _LO), f32),
        interpret=interpret,
    )(x, w1bd, b1f, w2blk, b2f, gsel, gates_t)

    combine = out.reshape(_B, _NG, _OC, _LO)
    gates_all = gates.transpose(1, 2, 0)
    return combine, loss[0, 0], gates_all


def kernel(x, Wg, W1, b1, W2, b2):
    return _run(x, Wg, W1, b1, W2, b2)


# speedup vs baseline: 1.0259x; 1.0259x over previous
import functools

import jax
import jax.numpy as jnp
import numpy as np
from jax.experimental import pallas as pl
from jax.experimental.pallas import tpu as pltpu

_NG = 4
_D = 32
_E = 8
_OC = 16
_B = 64
_L = 4096
_LO = _L - 2
_GD = _NG * _OC


def _cv2(v):
    mean = jnp.sum(v, axis=1, keepdims=True) / _E
    var = jnp.sum((v - mean) ** 2, axis=1, keepdims=True) / (_E - 1)
    return var / (mean * mean + 1e-10)


def _gate_body(gx_ref, wg_ref, gates_ref, loss_ref):
    iota = jax.lax.broadcasted_iota(jnp.int32, (_B, _E), 1)
    loss = jnp.zeros((1, 1), jnp.float32)
    for g in range(_NG):
        logits = jnp.dot(gx_ref[g], wg_ref[g],
                         preferred_element_type=jnp.float32)
        p = jax.nn.softmax(logits, axis=1)
        m1 = jnp.max(p, axis=1, keepdims=True)
        i1 = jnp.min(jnp.where(p == m1, iota, _E), axis=1, keepdims=True)
        mask1 = iota == i1
        p2 = jnp.where(mask1, -jnp.inf, p)
        m2 = jnp.max(p2, axis=1, keepdims=True)
        i2 = jnp.min(jnp.where(p2 == m2, iota, _E), axis=1, keepdims=True)
        denom = m1 + m2 + 1e-6
        gates = jnp.where(mask1, m1 / denom, 0.0) + jnp.where(
            iota == i2, m2 / denom, 0.0)
        gates_ref[g] = gates
        imp = jnp.sum(gates, axis=0, keepdims=True)
        load = jnp.sum((gates > 0.0).astype(jnp.float32), axis=0,
                       keepdims=True)
        loss = loss + (_cv2(imp) + _cv2(load)) * 0.01
    loss_ref[...] = loss


_BB = 4


def _main_body(x_ref, w1_ref, b1_ref, w2blk_ref, b2f_ref, gsel_ref, g_ref,
               out_ref):
    for bb in range(_BB):
        x = x_ref[bb].astype(jnp.bfloat16)
        w1 = w1_ref[...].astype(jnp.bfloat16)
        y0 = jnp.dot(w1[0], x, preferred_element_type=jnp.float32)
        y1 = jnp.dot(w1[1], x, preferred_element_type=jnp.float32)
        y2 = jnp.dot(w1[2], x, preferred_element_type=jnp.float32)
        acc = (y0[:, 0:_LO] + y1[:, 1:1 + _LO] + y2[:, 2:2 + _LO]
               + b1_ref[...])
        h = jnp.tanh(acc)
        gb = g_ref[bb]
        ge = jnp.dot(gsel_ref[...], gb, preferred_element_type=jnp.float32)
        w2bd = jnp.zeros((_GD, _GD), jnp.float32)
        for e in range(_E):
            w2bd = w2bd + w2blk_ref[e] * ge[:, e:e + 1]
        b2e = jnp.sum(b2f_ref[...] * ge, axis=1, keepdims=True)
        out = jnp.dot(w2bd, h, preferred_element_type=jnp.float32) + b2e
        out_ref[bb] = out


@functools.partial(jax.jit, static_argnames=("interpret",))
def _run(x, Wg, W1, b1, W2, b2, interpret=False):
    f32 = jnp.float32
    xs = jax.lax.slice_in_dim(x, _L - 6, _L - 1, axis=2)
    gx = xs.reshape(_B, _NG, _D * 5).transpose(1, 0, 2)

    gates, loss = pl.pallas_call(
        _gate_body,
        out_shape=(
            jax.ShapeDtypeStruct((_NG, _B, _E), f32),
            jax.ShapeDtypeStruct((1, 1), f32),
        ),
        interpret=interpret,
    )(gx, Wg)

    w1bd = jnp.zeros((3, _GD, _NG * _D), f32)
    for g in range(_NG):
        w1bd = jax.lax.dynamic_update_slice(
            w1bd, W1[g].transpose(2, 0, 1), (0, g * _OC, g * _D))
    b1f = b1.reshape(_GD, 1)
    w2r = W2[:, :, :, 0].reshape(_NG, _OC, _E, _OC)
    w2blk = jnp.zeros((_E, _GD, _GD), f32)
    for g in range(_NG):
        w2blk = jax.lax.dynamic_update_slice(
            w2blk, w2r[g].transpose(1, 0, 2), (0, g * _OC, g * _OC))
    b2f = b2.reshape(_NG, _OC, _E).reshape(_GD, _E)
    gsel = jnp.repeat(jnp.eye(_NG, dtype=f32), _OC, axis=0)
    gates_t = gates.transpose(1, 0, 2)

    out = pl.pallas_call(
        _main_body,
        grid=(_B // _BB,),
        in_specs=[
            pl.BlockSpec((_BB, _NG * _D, _L), lambda b: (b, 0, 0)),
            pl.BlockSpec((3, _GD, _NG * _D), lambda b: (0, 0, 0)),
            pl.BlockSpec((_GD, 1), lambda b: (0, 0)),
            pl.BlockSpec((_E, _GD, _GD), lambda b: (0, 0, 0)),
            pl.BlockSpec((_GD, _E), lambda b: (0, 0)),
            pl.BlockSpec((_GD, _NG), lambda b: (0, 0)),
            pl.BlockSpec((_BB, _NG, _E), lambda b: (b, 0, 0)),
        ],
        out_specs=pl.BlockSpec((_BB, _GD, _LO), lambda b: (b, 0, 0)),
        out_shape=jax.ShapeDtypeStruct((_B, _GD, _LO), f32),
        interpret=interpret,
    )(x, w1bd, b1f, w2blk, b2f, gsel, gates_t)

    combine = out.reshape(_B, _NG, _OC, _LO)
    gates_all = gates.transpose(1, 2, 0)
    return combine, loss[0, 0], gates_all


def kernel(x, Wg, W1, b1, W2, b2):
    return _run(x, Wg, W1, b1, W2, b2)


# speedup vs baseline: 1.0616x; 1.0348x over previous
import functools

import jax
import jax.numpy as jnp
import numpy as np
from jax.experimental import pallas as pl
from jax.experimental.pallas import tpu as pltpu

_NG = 4
_D = 32
_E = 8
_OC = 16
_B = 64
_L = 4096
_LO = _L - 2
_GD = _NG * _OC


def _cv2(v):
    mean = jnp.sum(v, axis=1, keepdims=True) / _E
    var = jnp.sum((v - mean) ** 2, axis=1, keepdims=True) / (_E - 1)
    return var / (mean * mean + 1e-10)


def _gate_body(gx_ref, wg_ref, gates_ref, loss_ref):
    iota = jax.lax.broadcasted_iota(jnp.int32, (_B, _E), 1)
    loss = jnp.zeros((1, 1), jnp.float32)
    for g in range(_NG):
        logits = jnp.dot(gx_ref[g], wg_ref[g],
                         preferred_element_type=jnp.float32)
        p = jax.nn.softmax(logits, axis=1)
        m1 = jnp.max(p, axis=1, keepdims=True)
        i1 = jnp.min(jnp.where(p == m1, iota, _E), axis=1, keepdims=True)
        mask1 = iota == i1
        p2 = jnp.where(mask1, -jnp.inf, p)
        m2 = jnp.max(p2, axis=1, keepdims=True)
        i2 = jnp.min(jnp.where(p2 == m2, iota, _E), axis=1, keepdims=True)
        denom = m1 + m2 + 1e-6
        gates = jnp.where(mask1, m1 / denom, 0.0) + jnp.where(
            iota == i2, m2 / denom, 0.0)
        gates_ref[g] = gates
        imp = jnp.sum(gates, axis=0, keepdims=True)
        load = jnp.sum((gates > 0.0).astype(jnp.float32), axis=0,
                       keepdims=True)
        loss = loss + (_cv2(imp) + _cv2(load)) * 0.01
    loss_ref[...] = loss


_BB = 4


def _main_body(x_ref, w1_ref, b1_ref, w2blk_ref, b2f_ref, gsel_ref, g_ref,
               out_ref):
    for bb in range(_BB):
        x = x_ref[bb]
        y0 = jnp.dot(w1_ref[0], x, preferred_element_type=jnp.float32)
        y1 = jnp.dot(w1_ref[1], x, preferred_element_type=jnp.float32)
        y2 = jnp.dot(w1_ref[2], x, preferred_element_type=jnp.float32)
        acc = (y0 + pltpu.roll(y1, _L - 1, 1) + pltpu.roll(y2, _L - 2, 1)
               + b1_ref[...])
        h = jnp.tanh(acc)
        gb = g_ref[bb]
        ge = jnp.dot(gsel_ref[...], gb, preferred_element_type=jnp.float32)
        w2bd = jnp.zeros((_GD, _GD), jnp.float32)
        for e in range(_E):
            w2bd = w2bd + w2blk_ref[e] * ge[:, e:e + 1]
        b2e = jnp.sum(b2f_ref[...] * ge, axis=1, keepdims=True)
        out = jnp.dot(w2bd, h, preferred_element_type=jnp.float32) + b2e
        out_ref[bb] = out[:, 0:_LO]


@functools.partial(jax.jit, static_argnames=("interpret",))
def _run(x, Wg, W1, b1, W2, b2, interpret=False):
    f32 = jnp.float32
    xs = jax.lax.slice_in_dim(x, _L - 6, _L - 1, axis=2)
    gx = xs.reshape(_B, _NG, _D * 5).transpose(1, 0, 2)

    gates, loss = pl.pallas_call(
        _gate_body,
        out_shape=(
            jax.ShapeDtypeStruct((_NG, _B, _E), f32),
            jax.ShapeDtypeStruct((1, 1), f32),
        ),
        interpret=interpret,
    )(gx, Wg)

    w1bd = jnp.zeros((3, _GD, _NG * _D), f32)
    for g in range(_NG):
        w1bd = jax.lax.dynamic_update_slice(
            w1bd, W1[g].transpose(2, 0, 1), (0, g * _OC, g * _D))
    b1f = b1.reshape(_GD, 1)
    w2r = W2[:, :, :, 0].reshape(_NG, _OC, _E, _OC)
    w2blk = jnp.zeros((_E, _GD, _GD), f32)
    for g in range(_NG):
        w2blk = jax.lax.dynamic_update_slice(
            w2blk, w2r[g].transpose(1, 0, 2), (0, g * _OC, g * _OC))
    b2f = b2.reshape(_NG, _OC, _E).reshape(_GD, _E)
    gsel = jnp.repeat(jnp.eye(_NG, dtype=f32), _OC, axis=0)
    gates_t = gates.transpose(1, 0, 2)

    out = pl.pallas_call(
        _main_body,
        grid=(_B // _BB,),
        in_specs=[
            pl.BlockSpec((_BB, _NG * _D, _L), lambda b: (b, 0, 0)),
            pl.BlockSpec((3, _GD, _NG * _D), lambda b: (0, 0, 0)),
            pl.BlockSpec((_GD, 1), lambda b: (0, 0)),
            pl.BlockSpec((_E, _GD, _GD), lambda b: (0, 0, 0)),
            pl.BlockSpec((_GD, _E), lambda b: (0, 0)),
            pl.BlockSpec((_GD, _NG), lambda b: (0, 0)),
            pl.BlockSpec((_BB, _NG, _E), lambda b: (b, 0, 0)),
        ],
        out_specs=pl.BlockSpec((_BB, _GD, _LO), lambda b: (b, 0, 0)),
        out_shape=jax.ShapeDtypeStruct((_B, _GD, _LO), f32),
        interpret=interpret,
    )(x, w1bd, b1f, w2blk, b2f, gsel, gates_t)

    combine = out.reshape(_B, _NG, _OC, _LO)
    gates_all = gates.transpose(1, 2, 0)
    return combine, loss[0, 0], gates_all


def kernel(x, Wg, W1, b1, W2, b2):
    return _run(x, Wg, W1, b1, W2, b2)
